# baseline (speedup 1.0000x reference)
import math

import jax
import jax.numpy as jnp
from jax.experimental import pallas as pl

_NUM_CLASSES = 20
_CUTOFF = 1.0

_V = [math.exp(-(i - 2) ** 2 / 2.0) for i in range(5)]
_V = [x / sum(_V) for x in _V]


def _dot(m, x):
    return jax.lax.dot_general(
        m, x, (((1,), (0,)), ((), ())),
        precision=jax.lax.Precision.HIGHEST,
        preferred_element_type=jnp.float32)


def _body(dp_ref, lp_ref, out_ref):
    H, W = out_ref.shape[1], out_ref.shape[2]
    dp = dp_ref[0]
    lp = lp_ref[0]
    JH = H + 4

    S = [dp[:, 4 + j:4 + j + W] for j in range(-4, 5)]

    cols = jax.lax.broadcasted_iota(jnp.int32, (1, W), 1)
    CM = [jnp.where((cols + a >= 0) & (cols + a <= W - 1), 1.0, 0.0)
          for a in range(-2, 3)]

    SM = {}
    for a in range(-2, 3):
        for j in range(a - 2, a + 3):
            SM[(j, a)] = S[4 + j] * CM[2 + a]
    B = [SM[(a, a)][2:2 + JH] for a in range(-2, 3)]

    L = [lp[:, 2 + dw:2 + dw + W] for dw in range(-2, 3)]

    hh = jax.lax.broadcasted_iota(jnp.int32, (H, JH), 0)
    rr = jax.lax.broadcasted_iota(jnp.int32, (H, JH), 1)
    dlt = rr - hh
    rowok = (rr >= 2) & (rr < JH - 2)
    mb = jnp.where((dlt >= 0) & (dlt <= 4) & rowok, 1.0, 0.0)
    mg = jnp.zeros((H, JH), jnp.float32)
    for i in range(5):
        mg = jnp.where((dlt == i) & rowok, _V[i], mg)

    slots = []

    for dh in range(-2, 3):
        for dw in range(-2, 3):
            if dh == 0 and dw == 0:
                continue
            t = [jnp.abs(SM[(dw + a, a)][2 + dh:2 + dh + JH] - B[2 + a])
                 for a in range(-2, 3)]
            u0 = t[0] + t[4]
            u1 = t[1] + t[3]
            rb = (u0 + u1) + t[2]
            rg = _V[0] * u0 + (_V[1] * u1 + _V[2] * t[2])
            dist = _dot(mb, rb) - _dot(mg, rg)

            lab = L[2 + dw][2 + dh:2 + dh + H]
            key = (jax.lax.bitcast_convert_type(dist, jnp.int32)
                   & jnp.int32(-32)) | lab
            if len(slots) < 4:
                slots.append(key)
            else:
                ck = key
                for i in range(4):
                    nk = jnp.minimum(slots[i], ck)
                    ck = jnp.maximum(slots[i], ck)
                    slots[i] = nk

    cut = jnp.int32(0x3F800000)
    ls = [lp[2:2 + H, 2:2 + W]]
    for i in range(4):
        di = slots[i] & jnp.int32(-32)
        ls.append(jnp.where(di > cut, _NUM_CLASSES, slots[i] & 31))

    ones = jnp.ones_like(ls[0])
    cnt = [ones, ones, ones, ones, ones]
    for i in range(5):
        for j in range(i + 1, 5):
            e = jnp.where(ls[i] == ls[j], 1, 0)
            cnt[i] = cnt[i] + e
            cnt[j] = cnt[j] + e
    key = jnp.zeros_like(ls[0])
    for i in range(5):
        ki = jnp.where(ls[i] == _NUM_CLASSES, 0,
                       cnt[i] * 32 + (31 - ls[i]))
        key = jnp.maximum(key, ki)
    best = jnp.where(key > 0, 31 - (key & 31), 0)
    out_ref[0] = best


def kernel(depth, label):
    B, C, H, W = depth.shape
    d = depth[:, 0]
    dp = jnp.pad(d, ((0, 0), (4, 4), (4, 4)))
    lp = jnp.pad(label, ((0, 0), (2, 2), (2, 2)))
    return pl.pallas_call(
        _body,
        grid=(B,),
        in_specs=[
            pl.BlockSpec((1, H + 8, W + 8), lambda b: (b, 0, 0)),
            pl.BlockSpec((1, H + 4, W + 4), lambda b: (b, 0, 0)),
        ],
        out_specs=pl.BlockSpec((1, H, W), lambda b: (b, 0, 0)),
        out_shape=jax.ShapeDtypeStruct((B, H, W), jnp.int32),
    )(dp, lp)
